# baseline (speedup 1.0000x reference)
import functools

import jax
import jax.numpy as jnp
from jax import lax
from jax.experimental import pallas as pl
from jax.experimental.pallas import tpu as pltpu
from jax.experimental.pallas import tpu_sc as plsc

HIDDEN = 512
BATCH = 16384
LANES = 16
NC = 2
NS = 16
NW = NC * NS
B_PER_W = BATCH // NW
CH = 16
NSTEPS = B_PER_W // CH
NGROUPS = CH // LANES
HHALF = HIDDEN // 2
DHALF = HHALF // LANES

_mesh = plsc.VectorSubcoreMesh(core_axis_name="c", subcore_axis_name="s")


@functools.partial(
    pl.kernel,
    mesh=_mesh,
    out_type=jax.ShapeDtypeStruct((BATCH, HIDDEN), jnp.float32),
    scratch_types=[
        pltpu.VMEM((B_PER_W,), jnp.int32),
        pltpu.VMEM((2, HIDDEN), jnp.float32),
        pltpu.VMEM((2, CH, HIDDEN), jnp.float32),
        pltpu.SemaphoreType.DMA,
        pltpu.SemaphoreType.DMA,
    ],
)
def _embed_lookup_sc(ids_hbm, w_hbm, out_hbm, idx_v, wbuf, obuf, sem0, sem1):
    wid = lax.axis_index("s") * NC + lax.axis_index("c")
    base = wid * B_PER_W
    c_ids = pltpu.async_copy(ids_hbm.at[pl.ds(base, B_PER_W)], idx_v, sem0)
    c_w = pltpu.async_copy(w_hbm, wbuf, sem1)
    c_ids.wait()
    c_w.wait()
    sems = (sem0, sem1)

    def _chunk_pair(p, carry):
        for b in range(2):
            s = p * 2 + b
            @pl.when(p > 0)
            def _drain(b=b):
                pltpu.make_async_copy(
                    obuf.at[b], out_hbm.at[pl.ds(base, CH)], sems[b]
                ).wait()

            for h in range(2):
                w0 = [lax.bitcast_convert_type(
                          wbuf[0, pl.ds(h * HHALF + d * LANES, LANES)],
                          jnp.int32)
                      for d in range(DHALF)]
                w1 = [lax.bitcast_convert_type(
                          wbuf[1, pl.ds(h * HHALF + d * LANES, LANES)],
                          jnp.int32)
                      for d in range(DHALF)]

                def _group(g, c, b=b, h=h, w0=w0, w1=w1, s=s):
                    ids16 = idx_v[pl.ds(s * LANES * NGROUPS + g * LANES,
                                        LANES)]
                    for j in range(LANES):
                        m = jnp.broadcast_to(-ids16[j], (LANES,))
                        nm = ~m
                        for d in range(DHALF):
                            obuf[b, g * LANES + j,
                                 pl.ds(h * HHALF + d * LANES, LANES)] = (
                                lax.bitcast_convert_type(
                                    (w1[d] & m) | (w0[d] & nm), jnp.float32))
                    return c

                lax.fori_loop(0, NGROUPS, _group, 0)
            pltpu.async_copy(
                obuf.at[b], out_hbm.at[pl.ds(base + s * CH, CH)], sems[b]
            )
        return carry

    lax.fori_loop(0, NSTEPS // 2, _chunk_pair, 0)
    for b in range(2):
        pltpu.make_async_copy(
            obuf.at[b], out_hbm.at[pl.ds(base, CH)], sems[b]
        ).wait()


def kernel(domain_ids, embed_weight):
    return _embed_lookup_sc(domain_ids.astype(jnp.int32), embed_weight)


# speedup vs baseline: 1.0093x; 1.0093x over previous
import functools

import jax
import jax.numpy as jnp
from jax import lax
from jax.experimental import pallas as pl
from jax.experimental.pallas import tpu as pltpu
from jax.experimental.pallas import tpu_sc as plsc

HIDDEN = 512
BATCH = 16384
LANES = 16
NC = 2
NS = 16
NW = NC * NS
B_PER_W = BATCH // NW
CH = 32
NSTEPS = B_PER_W // CH
NGROUPS = CH // LANES
HHALF = HIDDEN // 2
DHALF = HHALF // LANES

_mesh = plsc.VectorSubcoreMesh(core_axis_name="c", subcore_axis_name="s")


@functools.partial(
    pl.kernel,
    mesh=_mesh,
    out_type=jax.ShapeDtypeStruct((BATCH, HIDDEN), jnp.float32),
    scratch_types=[
        pltpu.VMEM((B_PER_W,), jnp.int32),
        pltpu.VMEM((2, HIDDEN), jnp.float32),
        pltpu.VMEM((2, CH, HIDDEN), jnp.float32),
        pltpu.SemaphoreType.DMA,
        pltpu.SemaphoreType.DMA,
    ],
)
def _embed_lookup_sc(ids_hbm, w_hbm, out_hbm, idx_v, wbuf, obuf, sem0, sem1):
    wid = lax.axis_index("s") * NC + lax.axis_index("c")
    base = wid * B_PER_W
    c_ids = pltpu.async_copy(ids_hbm.at[pl.ds(base, B_PER_W)], idx_v, sem0)
    c_w = pltpu.async_copy(w_hbm, wbuf, sem1)
    c_ids.wait()
    c_w.wait()
    sems = (sem0, sem1)

    def _chunk_pair(p, carry):
        for b in range(2):
            s = p * 2 + b
            @pl.when(p > 0)
            def _drain(b=b):
                pltpu.make_async_copy(
                    obuf.at[b], out_hbm.at[pl.ds(base, CH)], sems[b]
                ).wait()

            for h in range(2):
                w0 = [lax.bitcast_convert_type(
                          wbuf[0, pl.ds(h * HHALF + d * LANES, LANES)],
                          jnp.int32)
                      for d in range(DHALF)]
                w1 = [lax.bitcast_convert_type(
                          wbuf[1, pl.ds(h * HHALF + d * LANES, LANES)],
                          jnp.int32)
                      for d in range(DHALF)]

                def _group(g, c, b=b, h=h, w0=w0, w1=w1, s=s):
                    ids16 = idx_v[pl.ds(s * LANES * NGROUPS + g * LANES,
                                        LANES)]
                    for j in range(LANES):
                        m = jnp.broadcast_to(-ids16[j], (LANES,))
                        nm = ~m
                        for d in range(DHALF):
                            obuf[b, g * LANES + j,
                                 pl.ds(h * HHALF + d * LANES, LANES)] = (
                                lax.bitcast_convert_type(
                                    (w1[d] & m) | (w0[d] & nm), jnp.float32))
                    return c

                lax.fori_loop(0, NGROUPS, _group, 0)
            pltpu.async_copy(
                obuf.at[b], out_hbm.at[pl.ds(base + s * CH, CH)], sems[b]
            )
        return carry

    lax.fori_loop(0, NSTEPS // 2, _chunk_pair, 0)
    for b in range(2):
        pltpu.make_async_copy(
            obuf.at[b], out_hbm.at[pl.ds(base, CH)], sems[b]
        ).wait()


def kernel(domain_ids, embed_weight):
    return _embed_lookup_sc(domain_ids.astype(jnp.int32), embed_weight)
